# baseline (speedup 1.0000x reference)
import functools

import jax
import jax.numpy as jnp
from jax import lax
from jax.experimental import pallas as pl
from jax.experimental.pallas import tpu as pltpu
from jax.experimental.pallas import tpu_sc as plsc

N = 10000
NP = 10240
E = 320000
D = 128
D_DENSE = 256
D_OUT = 64
NC = 2
NS = 16
NW = NC * NS
EPW = E // NW
C = 125
NCHUNK = EPW // C
SLAB = NP // NS

_mesh = dict(core_axis_name="c", subcore_axis_name="s")


@functools.partial(
    pl.kernel,
    out_type=jax.ShapeDtypeStruct((NC, NP), jnp.float32),
    mesh=plsc.VectorSubcoreMesh(**_mesh),
    scratch_types=[
        pltpu.VMEM((NCHUNK, C), jnp.int32),
        pltpu.VMEM((128,), jnp.float32),
        pltpu.VMEM_SHARED((NP,), jnp.float32),
        pltpu.SemaphoreType.DMA,
        pltpu.SemaphoreType.DMA,
    ],
)
def _deg_parts(dstpk_hbm, zrow_hbm, out_hbm, dsti, onesb, acc, ss0, ss1):
    cid = lax.axis_index("c")
    sid = lax.axis_index("s")
    wid = sid * NC + cid
    ones16 = jnp.ones((16,), jnp.float32)
    for j in range(8):
        onesb[pl.ds(j * 16, 16)] = ones16
    ones = onesb.at[pl.ds(0, C)]
    pltpu.sync_copy(dstpk_hbm.at[wid], dsti)
    col0 = sid * SLAB
    pltpu.sync_copy(zrow_hbm, acc.at[pl.ds(col0, SLAB)])
    plsc.subcore_barrier()

    pltpu.async_copy(ones, acc.at[dsti.at[0]], ss0, add=True)
    pltpu.async_copy(ones, acc.at[dsti.at[1]], ss1, add=True)

    def cbody(t, _):
        i = 2 * t
        pltpu.make_async_copy(ones, acc.at[dsti.at[i]], ss0).wait()
        pltpu.async_copy(ones, acc.at[dsti.at[i + 2]], ss0, add=True)
        pltpu.make_async_copy(ones, acc.at[dsti.at[i + 1]], ss1).wait()
        pltpu.async_copy(ones, acc.at[dsti.at[i + 3]], ss1, add=True)
        return 0
    lax.fori_loop(0, NCHUNK // 2 - 1, cbody, 0)
    pltpu.make_async_copy(ones, acc.at[dsti.at[NCHUNK - 2]], ss0).wait()
    pltpu.make_async_copy(ones, acc.at[dsti.at[NCHUNK - 1]], ss1).wait()

    plsc.subcore_barrier()
    pltpu.sync_copy(acc.at[pl.ds(col0, SLAB)], out_hbm.at[cid, pl.ds(col0, SLAB)])


@functools.partial(
    pl.kernel,
    out_type=jax.ShapeDtypeStruct((NC * NP, D), jnp.float32),
    mesh=plsc.VectorSubcoreMesh(**_mesh),
    scratch_types=[
        pltpu.VMEM((NCHUNK, C), jnp.int32),
        pltpu.VMEM((C,), jnp.int32),
        pltpu.VMEM((C,), jnp.int32),
        pltpu.VMEM((C, D), jnp.float32),
        pltpu.VMEM((C, D), jnp.float32),
        pltpu.VMEM_SHARED((NP, D), jnp.float32),
        pltpu.SemaphoreType.DMA,
        pltpu.SemaphoreType.DMA,
        pltpu.SemaphoreType.DMA,
        pltpu.SemaphoreType.DMA,
        pltpu.SemaphoreType.DMA,
        pltpu.SemaphoreType.DMA,
    ],
)
def _edge_agg(hs_hbm, srcpk_hbm, dstpk_hbm, zslab_hbm, out_hbm,
              dsti, srcb0, srcb1, rows0, rows1, acc, gs0, gs1, ss0, ss1, is0, is1):
    cid = lax.axis_index("c")
    sid = lax.axis_index("s")
    wid = sid * NC + cid
    r0 = sid * SLAB
    pltpu.sync_copy(dstpk_hbm.at[wid], dsti)
    pltpu.sync_copy(srcpk_hbm.at[wid, 0], srcb0)
    pltpu.async_copy(hs_hbm.at[srcb0], rows0, gs0)
    pltpu.sync_copy(srcpk_hbm.at[wid, 1], srcb1)
    pltpu.async_copy(hs_hbm.at[srcb1], rows1, gs1)
    pltpu.sync_copy(zslab_hbm, acc.at[pl.ds(r0, SLAB)])
    plsc.subcore_barrier()

    def cbody(t, _):
        i = 2 * t
        pltpu.make_async_copy(hs_hbm.at[srcb0], rows0, gs0).wait()
        pltpu.async_copy(srcpk_hbm.at[wid, i + 2], srcb0, is0)
        pltpu.async_copy(rows0, acc.at[dsti.at[i]], ss0, add=True)
        pltpu.make_async_copy(rows0, acc.at[dsti.at[i]], ss0).wait()
        pltpu.make_async_copy(srcpk_hbm.at[wid, i + 2], srcb0, is0).wait()
        pltpu.async_copy(hs_hbm.at[srcb0], rows0, gs0)
        pltpu.make_async_copy(hs_hbm.at[srcb1], rows1, gs1).wait()
        pltpu.async_copy(srcpk_hbm.at[wid, i + 3], srcb1, is1)
        pltpu.async_copy(rows1, acc.at[dsti.at[i + 1]], ss1, add=True)
        pltpu.make_async_copy(rows1, acc.at[dsti.at[i + 1]], ss1).wait()
        pltpu.make_async_copy(srcpk_hbm.at[wid, i + 3], srcb1, is1).wait()
        pltpu.async_copy(hs_hbm.at[srcb1], rows1, gs1)
        return 0
    lax.fori_loop(0, NCHUNK // 2 - 1, cbody, 0)

    i = NCHUNK - 2
    pltpu.make_async_copy(hs_hbm.at[srcb0], rows0, gs0).wait()
    pltpu.async_copy(rows0, acc.at[dsti.at[i]], ss0, add=True)
    pltpu.make_async_copy(rows0, acc.at[dsti.at[i]], ss0).wait()
    pltpu.make_async_copy(hs_hbm.at[srcb1], rows1, gs1).wait()
    pltpu.async_copy(rows1, acc.at[dsti.at[i + 1]], ss1, add=True)
    pltpu.make_async_copy(rows1, acc.at[dsti.at[i + 1]], ss1).wait()

    plsc.subcore_barrier()
    pltpu.sync_copy(acc.at[pl.ds(r0, SLAB)], out_hbm.at[pl.ds(cid * NP + r0, SLAB)])


R = 1024
GRID = NP // R


def _dis_body(parts_ref, out_ref):
    p = parts_ref[...]
    out_ref[...] = lax.rsqrt(p[0:1, :] + p[1:2, :] + 1.0)


_dis_call = pl.pallas_call(
    _dis_body,
    out_shape=jax.ShapeDtypeStruct((1, NP), jnp.float32),
)


def _mm_body(x_ref, w_ref, hw_ref):
    hw_ref[...] = jnp.dot(x_ref[...], w_ref[...], preferred_element_type=jnp.float32)


_mm_call = pl.pallas_call(
    _mm_body,
    grid=(GRID,),
    in_specs=[
        pl.BlockSpec((R, D), lambda i: (i, 0)),
        pl.BlockSpec((D, D), lambda i: (0, 0)),
    ],
    out_specs=pl.BlockSpec((R, D), lambda i: (i, 0)),
    out_shape=jax.ShapeDtypeStruct((NP, D), jnp.float32),
)


def _scale_body(hw_ref, dis_ref, hs_ref):
    hs_ref[...] = hw_ref[...] * dis_ref[...]


_scale_call = pl.pallas_call(
    _scale_body,
    grid=(GRID,),
    in_specs=[
        pl.BlockSpec((R, D), lambda i: (i, 0)),
        pl.BlockSpec((R, 1), lambda i: (i, 0)),
    ],
    out_specs=pl.BlockSpec((R, D), lambda i: (i, 0)),
    out_shape=jax.ShapeDtypeStruct((NP, D), jnp.float32),
)


def _ln_relu(conv, g, be):
    a = jnp.maximum(conv, 0.0)
    m = jnp.mean(a, axis=-1, keepdims=True)
    v = jnp.mean((a - m) ** 2, axis=-1, keepdims=True)
    return (a - m) * lax.rsqrt(v + 1e-5) * g + be


def _layer_body(hw_ref, p0_ref, p1_ref, dis_ref, b_ref, g_ref, be_ref, wn_ref,
                hwn_ref, hsn_ref):
    dis = dis_ref[...]
    conv = dis * (p0_ref[...] + p1_ref[...]) + (dis * dis) * hw_ref[...] + b_ref[...]
    h = _ln_relu(conv, g_ref[...], be_ref[...])
    hwn = jnp.dot(h, wn_ref[...], preferred_element_type=jnp.float32)
    hwn_ref[...] = hwn
    hsn_ref[...] = hwn * dis


_layer_call = pl.pallas_call(
    _layer_body,
    grid=(GRID,),
    in_specs=[
        pl.BlockSpec((R, D), lambda i: (i, 0)),
        pl.BlockSpec((R, D), lambda i: (i, 0)),
        pl.BlockSpec((R, D), lambda i: (NP // R + i, 0)),
        pl.BlockSpec((R, 1), lambda i: (i, 0)),
        pl.BlockSpec((1, D), lambda i: (0, 0)),
        pl.BlockSpec((1, D), lambda i: (0, 0)),
        pl.BlockSpec((1, D), lambda i: (0, 0)),
        pl.BlockSpec((D, D), lambda i: (0, 0)),
    ],
    out_specs=[pl.BlockSpec((R, D), lambda i: (i, 0))] * 2,
    out_shape=[jax.ShapeDtypeStruct((NP, D), jnp.float32)] * 2,
)


def _final_body(hw_ref, p0_ref, p1_ref, dis_ref, b_ref, g_ref, be_ref,
                wp1_ref, bp1_ref, wp2_ref, bp2_ref, wp3_ref, bp3_ref,
                emb_ref, logp_ref):
    dis = dis_ref[...]
    conv = dis * (p0_ref[...] + p1_ref[...]) + (dis * dis) * hw_ref[...] + b_ref[...]
    emb_ref[...] = conv
    h = _ln_relu(conv, g_ref[...], be_ref[...])
    h = jnp.dot(h, wp1_ref[...], preferred_element_type=jnp.float32) + bp1_ref[...]
    h = jnp.dot(h, wp2_ref[...], preferred_element_type=jnp.float32) + bp2_ref[...]
    h = jnp.dot(h, wp3_ref[...], preferred_element_type=jnp.float32) + bp3_ref[...]
    mx = jnp.max(h, axis=-1, keepdims=True)
    sh = h - mx
    lse = jnp.log(jnp.sum(jnp.exp(sh), axis=-1, keepdims=True))
    logp_ref[...] = sh - lse


_final_call = pl.pallas_call(
    _final_body,
    grid=(GRID,),
    in_specs=[
        pl.BlockSpec((R, D), lambda i: (i, 0)),
        pl.BlockSpec((R, D), lambda i: (i, 0)),
        pl.BlockSpec((R, D), lambda i: (NP // R + i, 0)),
        pl.BlockSpec((R, 1), lambda i: (i, 0)),
        pl.BlockSpec((1, D), lambda i: (0, 0)),
        pl.BlockSpec((1, D), lambda i: (0, 0)),
        pl.BlockSpec((1, D), lambda i: (0, 0)),
        pl.BlockSpec((D, D_DENSE), lambda i: (0, 0)),
        pl.BlockSpec((1, D_DENSE), lambda i: (0, 0)),
        pl.BlockSpec((D_DENSE, D), lambda i: (0, 0)),
        pl.BlockSpec((1, D), lambda i: (0, 0)),
        pl.BlockSpec((D, D_OUT), lambda i: (0, 0)),
        pl.BlockSpec((1, D_OUT), lambda i: (0, 0)),
    ],
    out_specs=[
        pl.BlockSpec((R, D), lambda i: (i, 0)),
        pl.BlockSpec((R, D_OUT), lambda i: (i, 0)),
    ],
    out_shape=[
        jax.ShapeDtypeStruct((NP, D), jnp.float32),
        jax.ShapeDtypeStruct((NP, D_OUT), jnp.float32),
    ],
)


def kernel(x, edge_index, edge_attr, batch, W1, b1, g1, be1, W2, b2, g2, be2,
           W3, b3, g3, be3, Wp1, bp1, Wp2, bp2, Wp3, bp3):
    src_pk = edge_index[0].reshape(NW, NCHUNK, C)
    dst_pk = edge_index[1].reshape(NW, NCHUNK, C)

    deg_parts = _deg_parts(dst_pk, jnp.zeros((SLAB,), jnp.float32))

    xp = jnp.pad(x, ((0, NP - N), (0, 0)))
    zslab = jnp.zeros((SLAB, D), jnp.float32)

    b1r, g1r, be1r = b1.reshape(1, D), g1.reshape(1, D), be1.reshape(1, D)
    b2r, g2r, be2r = b2.reshape(1, D), g2.reshape(1, D), be2.reshape(1, D)
    b3r, g3r, be3r = b3.reshape(1, D), g3.reshape(1, D), be3.reshape(1, D)

    hW = _mm_call(xp, W1)
    dis = _dis_call(deg_parts).reshape(NP, 1)
    hs = _scale_call(hW, dis)
    parts = _edge_agg(hs, src_pk, dst_pk, zslab)
    hW, hs = _layer_call(hW, parts, parts, dis, b1r, g1r, be1r, W2)
    parts = _edge_agg(hs, src_pk, dst_pk, zslab)
    hW, hs = _layer_call(hW, parts, parts, dis, b2r, g2r, be2r, W3)
    parts = _edge_agg(hs, src_pk, dst_pk, zslab)
    emb, logp = _final_call(
        hW, parts, parts, dis, b3r, g3r, be3r,
        Wp1, bp1.reshape(1, D_DENSE), Wp2, bp2.reshape(1, D),
        Wp3, bp3.reshape(1, D_OUT))
    return emb[:N], logp[:N]


# speedup vs baseline: 1.0312x; 1.0312x over previous
import functools

import jax
import jax.numpy as jnp
from jax import lax
from jax.experimental import pallas as pl
from jax.experimental.pallas import tpu as pltpu
from jax.experimental.pallas import tpu_sc as plsc

N = 10000
NP = 10240
E = 320000
D = 128
D_DENSE = 256
D_OUT = 64
NC = 2
NS = 16
NW = NC * NS
EPW = E // NW
C = 125
NCHUNK = EPW // C
SLAB = NP // NS

_mesh = dict(core_axis_name="c", subcore_axis_name="s")


@functools.partial(
    pl.kernel,
    out_type=jax.ShapeDtypeStruct((NC, NP), jnp.float32),
    mesh=plsc.VectorSubcoreMesh(**_mesh),
    scratch_types=[
        pltpu.VMEM((NCHUNK, C), jnp.int32),
        pltpu.VMEM((128,), jnp.float32),
        pltpu.VMEM_SHARED((NP,), jnp.float32),
        pltpu.SemaphoreType.DMA,
        pltpu.SemaphoreType.DMA,
    ],
)
def _deg_parts(ei_hbm, zrow_hbm, out_hbm, dsti, onesb, acc, ss0, ss1):
    cid = lax.axis_index("c")
    sid = lax.axis_index("s")
    wid = sid * NC + cid
    ones16 = jnp.ones((16,), jnp.float32)
    for j in range(8):
        onesb[pl.ds(j * 16, 16)] = ones16
    ones = onesb.at[pl.ds(0, C)]
    pltpu.sync_copy(ei_hbm.at[1, wid], dsti)
    col0 = sid * SLAB
    pltpu.sync_copy(zrow_hbm, acc.at[pl.ds(col0, SLAB)])
    plsc.subcore_barrier()

    pltpu.async_copy(ones, acc.at[dsti.at[0]], ss0, add=True)
    pltpu.async_copy(ones, acc.at[dsti.at[1]], ss1, add=True)

    def cbody(t, _):
        i = 2 * t
        pltpu.make_async_copy(ones, acc.at[dsti.at[i]], ss0).wait()
        pltpu.async_copy(ones, acc.at[dsti.at[i + 2]], ss0, add=True)
        pltpu.make_async_copy(ones, acc.at[dsti.at[i + 1]], ss1).wait()
        pltpu.async_copy(ones, acc.at[dsti.at[i + 3]], ss1, add=True)
        return 0
    lax.fori_loop(0, NCHUNK // 2 - 1, cbody, 0)
    pltpu.make_async_copy(ones, acc.at[dsti.at[NCHUNK - 2]], ss0).wait()
    pltpu.make_async_copy(ones, acc.at[dsti.at[NCHUNK - 1]], ss1).wait()

    plsc.subcore_barrier()
    pltpu.sync_copy(acc.at[pl.ds(col0, SLAB)], out_hbm.at[cid, pl.ds(col0, SLAB)])


@functools.partial(
    pl.kernel,
    out_type=jax.ShapeDtypeStruct((NC * NP, D), jnp.float32),
    mesh=plsc.VectorSubcoreMesh(**_mesh),
    scratch_types=[
        pltpu.VMEM((NCHUNK, C), jnp.int32),
        pltpu.VMEM((C,), jnp.int32),
        pltpu.VMEM((C,), jnp.int32),
        pltpu.VMEM((C, D), jnp.float32),
        pltpu.VMEM((C, D), jnp.float32),
        pltpu.VMEM_SHARED((NP, D), jnp.float32),
        pltpu.SemaphoreType.DMA,
        pltpu.SemaphoreType.DMA,
        pltpu.SemaphoreType.DMA,
        pltpu.SemaphoreType.DMA,
        pltpu.SemaphoreType.DMA,
        pltpu.SemaphoreType.DMA,
    ],
)
def _edge_agg(hs_hbm, ei_hbm, zslab_hbm, out_hbm,
              dsti, srcb0, srcb1, rows0, rows1, acc, gs0, gs1, ss0, ss1, is0, is1):
    cid = lax.axis_index("c")
    sid = lax.axis_index("s")
    wid = sid * NC + cid
    r0 = sid * SLAB
    pltpu.sync_copy(ei_hbm.at[1, wid], dsti)
    pltpu.sync_copy(ei_hbm.at[0, wid, 0], srcb0)
    pltpu.async_copy(hs_hbm.at[srcb0], rows0, gs0)
    pltpu.sync_copy(ei_hbm.at[0, wid, 1], srcb1)
    pltpu.async_copy(hs_hbm.at[srcb1], rows1, gs1)
    pltpu.sync_copy(zslab_hbm, acc.at[pl.ds(r0, SLAB)])
    plsc.subcore_barrier()

    def cbody(t, _):
        i = 2 * t
        pltpu.make_async_copy(hs_hbm.at[srcb0], rows0, gs0).wait()
        pltpu.async_copy(ei_hbm.at[0, wid, i + 2], srcb0, is0)
        pltpu.async_copy(rows0, acc.at[dsti.at[i]], ss0, add=True)
        pltpu.make_async_copy(rows0, acc.at[dsti.at[i]], ss0).wait()
        pltpu.make_async_copy(ei_hbm.at[0, wid, i + 2], srcb0, is0).wait()
        pltpu.async_copy(hs_hbm.at[srcb0], rows0, gs0)
        pltpu.make_async_copy(hs_hbm.at[srcb1], rows1, gs1).wait()
        pltpu.async_copy(ei_hbm.at[0, wid, i + 3], srcb1, is1)
        pltpu.async_copy(rows1, acc.at[dsti.at[i + 1]], ss1, add=True)
        pltpu.make_async_copy(rows1, acc.at[dsti.at[i + 1]], ss1).wait()
        pltpu.make_async_copy(ei_hbm.at[0, wid, i + 3], srcb1, is1).wait()
        pltpu.async_copy(hs_hbm.at[srcb1], rows1, gs1)
        return 0
    lax.fori_loop(0, NCHUNK // 2 - 1, cbody, 0)

    i = NCHUNK - 2
    pltpu.make_async_copy(hs_hbm.at[srcb0], rows0, gs0).wait()
    pltpu.async_copy(rows0, acc.at[dsti.at[i]], ss0, add=True)
    pltpu.make_async_copy(rows0, acc.at[dsti.at[i]], ss0).wait()
    pltpu.make_async_copy(hs_hbm.at[srcb1], rows1, gs1).wait()
    pltpu.async_copy(rows1, acc.at[dsti.at[i + 1]], ss1, add=True)
    pltpu.make_async_copy(rows1, acc.at[dsti.at[i + 1]], ss1).wait()

    plsc.subcore_barrier()
    pltpu.sync_copy(acc.at[pl.ds(r0, SLAB)], out_hbm.at[pl.ds(cid * NP + r0, SLAB)])


R = 1000
GRID = N // R


def _dis_body(parts_ref, out_ref):
    p = parts_ref[...]
    out_ref[...] = jnp.transpose(lax.rsqrt(p[0:1, :] + p[1:2, :] + 1.0))


_dis_call = pl.pallas_call(
    _dis_body,
    out_shape=jax.ShapeDtypeStruct((NP, 1), jnp.float32),
)


def _stage0_body(x_ref, w_ref, dis_ref, hw_ref, hs_ref):
    hw = jnp.dot(x_ref[...], w_ref[...], preferred_element_type=jnp.float32)
    hw_ref[...] = hw
    hs_ref[...] = hw * dis_ref[...]


_stage0_call = pl.pallas_call(
    _stage0_body,
    grid=(GRID,),
    in_specs=[
        pl.BlockSpec((R, D), lambda i: (i, 0)),
        pl.BlockSpec((D, D), lambda i: (0, 0)),
        pl.BlockSpec((R, 1), lambda i: (i, 0)),
    ],
    out_specs=[pl.BlockSpec((R, D), lambda i: (i, 0))] * 2,
    out_shape=[jax.ShapeDtypeStruct((N, D), jnp.float32)] * 2,
)


def _ln_relu(conv, g, be):
    a = jnp.maximum(conv, 0.0)
    m = jnp.mean(a, axis=-1, keepdims=True)
    v = jnp.mean((a - m) ** 2, axis=-1, keepdims=True)
    return (a - m) * lax.rsqrt(v + 1e-5) * g + be


def _layer_body(hw_ref, p0_ref, p1_ref, dis_ref, b_ref, g_ref, be_ref, wn_ref,
                hwn_ref, hsn_ref):
    dis = dis_ref[...]
    agg = p0_ref[0] + p1_ref[0]
    conv = dis * agg + (dis * dis) * hw_ref[...] + b_ref[...]
    h = _ln_relu(conv, g_ref[...], be_ref[...])
    hwn = jnp.dot(h, wn_ref[...], preferred_element_type=jnp.float32)
    hwn_ref[...] = hwn
    hsn_ref[...] = hwn * dis


_layer_call = pl.pallas_call(
    _layer_body,
    grid=(GRID,),
    in_specs=[
        pl.BlockSpec((R, D), lambda i: (i, 0)),
        pl.BlockSpec((1, R, D), lambda i: (0, i, 0)),
        pl.BlockSpec((1, R, D), lambda i: (1, i, 0)),
        pl.BlockSpec((R, 1), lambda i: (i, 0)),
        pl.BlockSpec((1, D), lambda i: (0, 0)),
        pl.BlockSpec((1, D), lambda i: (0, 0)),
        pl.BlockSpec((1, D), lambda i: (0, 0)),
        pl.BlockSpec((D, D), lambda i: (0, 0)),
    ],
    out_specs=[pl.BlockSpec((R, D), lambda i: (i, 0))] * 2,
    out_shape=[jax.ShapeDtypeStruct((N, D), jnp.float32)] * 2,
)


def _final_body(hw_ref, p0_ref, p1_ref, dis_ref, b_ref, g_ref, be_ref,
                wp1_ref, bp1_ref, wp2_ref, bp2_ref, wp3_ref, bp3_ref,
                emb_ref, logp_ref):
    dis = dis_ref[...]
    agg = p0_ref[0] + p1_ref[0]
    conv = dis * agg + (dis * dis) * hw_ref[...] + b_ref[...]
    emb_ref[...] = conv
    h = _ln_relu(conv, g_ref[...], be_ref[...])
    h = jnp.dot(h, wp1_ref[...], preferred_element_type=jnp.float32) + bp1_ref[...]
    h = jnp.dot(h, wp2_ref[...], preferred_element_type=jnp.float32) + bp2_ref[...]
    h = jnp.dot(h, wp3_ref[...], preferred_element_type=jnp.float32) + bp3_ref[...]
    mx = jnp.max(h, axis=-1, keepdims=True)
    sh = h - mx
    lse = jnp.log(jnp.sum(jnp.exp(sh), axis=-1, keepdims=True))
    logp_ref[...] = sh - lse


_final_call = pl.pallas_call(
    _final_body,
    grid=(GRID,),
    in_specs=[
        pl.BlockSpec((R, D), lambda i: (i, 0)),
        pl.BlockSpec((1, R, D), lambda i: (0, i, 0)),
        pl.BlockSpec((1, R, D), lambda i: (1, i, 0)),
        pl.BlockSpec((R, 1), lambda i: (i, 0)),
        pl.BlockSpec((1, D), lambda i: (0, 0)),
        pl.BlockSpec((1, D), lambda i: (0, 0)),
        pl.BlockSpec((1, D), lambda i: (0, 0)),
        pl.BlockSpec((D, D_DENSE), lambda i: (0, 0)),
        pl.BlockSpec((1, D_DENSE), lambda i: (0, 0)),
        pl.BlockSpec((D_DENSE, D), lambda i: (0, 0)),
        pl.BlockSpec((1, D), lambda i: (0, 0)),
        pl.BlockSpec((D, D_OUT), lambda i: (0, 0)),
        pl.BlockSpec((1, D_OUT), lambda i: (0, 0)),
    ],
    out_specs=[
        pl.BlockSpec((R, D), lambda i: (i, 0)),
        pl.BlockSpec((R, D_OUT), lambda i: (i, 0)),
    ],
    out_shape=[
        jax.ShapeDtypeStruct((N, D), jnp.float32),
        jax.ShapeDtypeStruct((N, D_OUT), jnp.float32),
    ],
)


def kernel(x, edge_index, edge_attr, batch, W1, b1, g1, be1, W2, b2, g2, be2,
           W3, b3, g3, be3, Wp1, bp1, Wp2, bp2, Wp3, bp3):
    ei_pk = edge_index.reshape(2, NW, NCHUNK, C)

    deg_parts = _deg_parts(ei_pk, jnp.zeros((SLAB,), jnp.float32))
    dis = _dis_call(deg_parts)

    zslab = jnp.zeros((SLAB, D), jnp.float32)

    b1r, g1r, be1r = b1.reshape(1, D), g1.reshape(1, D), be1.reshape(1, D)
    b2r, g2r, be2r = b2.reshape(1, D), g2.reshape(1, D), be2.reshape(1, D)
    b3r, g3r, be3r = b3.reshape(1, D), g3.reshape(1, D), be3.reshape(1, D)

    hW, hs = _stage0_call(x, W1, dis)
    parts = _edge_agg(hs, ei_pk, zslab).reshape(NC, NP, D)
    hW, hs = _layer_call(hW, parts, parts, dis, b1r, g1r, be1r, W2)
    parts = _edge_agg(hs, ei_pk, zslab).reshape(NC, NP, D)
    hW, hs = _layer_call(hW, parts, parts, dis, b2r, g2r, be2r, W3)
    parts = _edge_agg(hs, ei_pk, zslab).reshape(NC, NP, D)
    emb, logp = _final_call(
        hW, parts, parts, dis, b3r, g3r, be3r,
        Wp1, bp1.reshape(1, D_DENSE), Wp2, bp2.reshape(1, D),
        Wp3, bp3.reshape(1, D_OUT))
    return emb, logp


# speedup vs baseline: 1.0490x; 1.0173x over previous
import functools

import jax
import jax.numpy as jnp
from jax import lax
from jax.experimental import pallas as pl
from jax.experimental.pallas import tpu as pltpu
from jax.experimental.pallas import tpu_sc as plsc

N = 10000
NP = 10240
E = 320000
D = 128
D_DENSE = 256
D_OUT = 64
NC = 2
NS = 16
NW = NC * NS
EPW = E // NW
C = 125
NCHUNK = EPW // C
SLAB = NP // NS

_mesh = dict(core_axis_name="c", subcore_axis_name="s")


@functools.partial(
    pl.kernel,
    out_type=jax.ShapeDtypeStruct((NC, NP), jnp.float32),
    mesh=plsc.VectorSubcoreMesh(**_mesh),
    scratch_types=[
        pltpu.VMEM((NCHUNK, C), jnp.int32),
        pltpu.VMEM((128,), jnp.float32),
        pltpu.VMEM_SHARED((NP,), jnp.float32),
        pltpu.SemaphoreType.DMA,
        pltpu.SemaphoreType.DMA,
    ],
)
def _deg_parts(ei_hbm, zrow_hbm, out_hbm, dsti, onesb, acc, ss0, ss1):
    cid = lax.axis_index("c")
    sid = lax.axis_index("s")
    wid = sid * NC + cid
    ones16 = jnp.ones((16,), jnp.float32)
    for j in range(8):
        onesb[pl.ds(j * 16, 16)] = ones16
    ones = onesb.at[pl.ds(0, C)]
    pltpu.sync_copy(ei_hbm.at[1, wid], dsti)
    col0 = sid * SLAB
    pltpu.sync_copy(zrow_hbm, acc.at[pl.ds(col0, SLAB)])
    plsc.subcore_barrier()

    pltpu.async_copy(ones, acc.at[dsti.at[0]], ss0, add=True)
    pltpu.async_copy(ones, acc.at[dsti.at[1]], ss1, add=True)

    def cbody(t, _):
        i = 2 * t
        pltpu.make_async_copy(ones, acc.at[dsti.at[i]], ss0).wait()
        pltpu.async_copy(ones, acc.at[dsti.at[i + 2]], ss0, add=True)
        pltpu.make_async_copy(ones, acc.at[dsti.at[i + 1]], ss1).wait()
        pltpu.async_copy(ones, acc.at[dsti.at[i + 3]], ss1, add=True)
        return 0
    lax.fori_loop(0, NCHUNK // 2 - 1, cbody, 0)
    pltpu.make_async_copy(ones, acc.at[dsti.at[NCHUNK - 2]], ss0).wait()
    pltpu.make_async_copy(ones, acc.at[dsti.at[NCHUNK - 1]], ss1).wait()

    plsc.subcore_barrier()
    pltpu.sync_copy(acc.at[pl.ds(col0, SLAB)], out_hbm.at[cid, pl.ds(col0, SLAB)])


@functools.partial(
    pl.kernel,
    out_type=jax.ShapeDtypeStruct((NC * NP, D), jnp.float32),
    mesh=plsc.VectorSubcoreMesh(**_mesh),
    scratch_types=[
        pltpu.VMEM((NCHUNK, C), jnp.int32),
        pltpu.VMEM((C,), jnp.int32),
        pltpu.VMEM((C,), jnp.int32),
        pltpu.VMEM((C, D), jnp.float32),
        pltpu.VMEM((C, D), jnp.float32),
        pltpu.VMEM_SHARED((NP, D), jnp.float32),
        pltpu.SemaphoreType.DMA,
        pltpu.SemaphoreType.DMA,
        pltpu.SemaphoreType.DMA,
        pltpu.SemaphoreType.DMA,
        pltpu.SemaphoreType.DMA,
        pltpu.SemaphoreType.DMA,
    ],
)
def _edge_agg(hs_hbm, ei_hbm, zslab_hbm, out_hbm,
              dsti, srcb0, srcb1, rows0, rows1, acc, gs0, gs1, ss0, ss1, is0, is1):
    cid = lax.axis_index("c")
    sid = lax.axis_index("s")
    wid = sid * NC + cid
    r0 = sid * SLAB
    pltpu.sync_copy(ei_hbm.at[1, wid], dsti)
    pltpu.sync_copy(ei_hbm.at[0, wid, 0], srcb0)
    pltpu.async_copy(hs_hbm.at[srcb0], rows0, gs0)
    pltpu.sync_copy(ei_hbm.at[0, wid, 1], srcb1)
    pltpu.async_copy(hs_hbm.at[srcb1], rows1, gs1)
    pltpu.sync_copy(zslab_hbm, acc.at[pl.ds(r0, SLAB)])
    plsc.subcore_barrier()

    def cbody(t, _):
        i = 2 * t
        pltpu.make_async_copy(hs_hbm.at[srcb0], rows0, gs0).wait()
        pltpu.async_copy(ei_hbm.at[0, wid, i + 2], srcb0, is0)
        pltpu.async_copy(rows0, acc.at[dsti.at[i]], ss0, add=True)
        pltpu.make_async_copy(rows0, acc.at[dsti.at[i]], ss0).wait()
        pltpu.make_async_copy(ei_hbm.at[0, wid, i + 2], srcb0, is0).wait()
        pltpu.async_copy(hs_hbm.at[srcb0], rows0, gs0)
        pltpu.make_async_copy(hs_hbm.at[srcb1], rows1, gs1).wait()
        pltpu.async_copy(ei_hbm.at[0, wid, i + 3], srcb1, is1)
        pltpu.async_copy(rows1, acc.at[dsti.at[i + 1]], ss1, add=True)
        pltpu.make_async_copy(rows1, acc.at[dsti.at[i + 1]], ss1).wait()
        pltpu.make_async_copy(ei_hbm.at[0, wid, i + 3], srcb1, is1).wait()
        pltpu.async_copy(hs_hbm.at[srcb1], rows1, gs1)
        return 0
    lax.fori_loop(0, NCHUNK // 2 - 1, cbody, 0)

    i = NCHUNK - 2
    pltpu.make_async_copy(hs_hbm.at[srcb0], rows0, gs0).wait()
    pltpu.async_copy(rows0, acc.at[dsti.at[i]], ss0, add=True)
    pltpu.make_async_copy(rows0, acc.at[dsti.at[i]], ss0).wait()
    pltpu.make_async_copy(hs_hbm.at[srcb1], rows1, gs1).wait()
    pltpu.async_copy(rows1, acc.at[dsti.at[i + 1]], ss1, add=True)
    pltpu.make_async_copy(rows1, acc.at[dsti.at[i + 1]], ss1).wait()

    plsc.subcore_barrier()
    pltpu.sync_copy(acc.at[pl.ds(r0, SLAB)], out_hbm.at[pl.ds(cid * NP + r0, SLAB)])


R = 1000
GRID = N // R


def _dis_body(parts_ref, out_ref):
    p = parts_ref[...]
    out_ref[...] = jnp.transpose(lax.rsqrt(p[0:1, :] + p[1:2, :] + 1.0))


_dis_call = pl.pallas_call(
    _dis_body,
    out_shape=jax.ShapeDtypeStruct((NP, 1), jnp.float32),
)


def _stage0_body(x_ref, w_ref, dis_ref, hs_ref):
    hw = jnp.dot(x_ref[...], w_ref[...], preferred_element_type=jnp.float32)
    hs_ref[...] = hw * dis_ref[...]


_stage0_call = pl.pallas_call(
    _stage0_body,
    grid=(GRID,),
    in_specs=[
        pl.BlockSpec((R, D), lambda i: (i, 0)),
        pl.BlockSpec((D, D), lambda i: (0, 0)),
        pl.BlockSpec((R, 1), lambda i: (i, 0)),
    ],
    out_specs=pl.BlockSpec((R, D), lambda i: (i, 0)),
    out_shape=jax.ShapeDtypeStruct((N, D), jnp.float32),
)


def _ln_relu(conv, g, be):
    a = jnp.maximum(conv, 0.0)
    m = jnp.mean(a, axis=-1, keepdims=True)
    v = jnp.mean((a - m) ** 2, axis=-1, keepdims=True)
    return (a - m) * lax.rsqrt(v + 1e-5) * g + be


def _layer_body(hs_ref, p0_ref, p1_ref, dis_ref, b_ref, g_ref, be_ref, wn_ref,
                hsn_ref):
    dis = dis_ref[...]
    agg = p0_ref[0] + p1_ref[0]
    conv = dis * (agg + hs_ref[...]) + b_ref[...]
    h = _ln_relu(conv, g_ref[...], be_ref[...])
    hwn = jnp.dot(h, wn_ref[...], preferred_element_type=jnp.float32)
    hsn_ref[...] = hwn * dis


_layer_call = pl.pallas_call(
    _layer_body,
    grid=(GRID,),
    in_specs=[
        pl.BlockSpec((R, D), lambda i: (i, 0)),
        pl.BlockSpec((1, R, D), lambda i: (0, i, 0)),
        pl.BlockSpec((1, R, D), lambda i: (1, i, 0)),
        pl.BlockSpec((R, 1), lambda i: (i, 0)),
        pl.BlockSpec((1, D), lambda i: (0, 0)),
        pl.BlockSpec((1, D), lambda i: (0, 0)),
        pl.BlockSpec((1, D), lambda i: (0, 0)),
        pl.BlockSpec((D, D), lambda i: (0, 0)),
    ],
    out_specs=pl.BlockSpec((R, D), lambda i: (i, 0)),
    out_shape=jax.ShapeDtypeStruct((N, D), jnp.float32),
)


def _final_body(hs_ref, p0_ref, p1_ref, dis_ref, b_ref, g_ref, be_ref,
                wp1_ref, bp1_ref, wp2_ref, bp2_ref, wp3_ref, bp3_ref,
                emb_ref, logp_ref):
    dis = dis_ref[...]
    agg = p0_ref[0] + p1_ref[0]
    conv = dis * (agg + hs_ref[...]) + b_ref[...]
    emb_ref[...] = conv
    h = _ln_relu(conv, g_ref[...], be_ref[...])
    h = jnp.dot(h, wp1_ref[...], preferred_element_type=jnp.float32) + bp1_ref[...]
    h = jnp.dot(h, wp2_ref[...], preferred_element_type=jnp.float32) + bp2_ref[...]
    h = jnp.dot(h, wp3_ref[...], preferred_element_type=jnp.float32) + bp3_ref[...]
    mx = jnp.max(h, axis=-1, keepdims=True)
    sh = h - mx
    lse = jnp.log(jnp.sum(jnp.exp(sh), axis=-1, keepdims=True))
    logp_ref[...] = sh - lse


_final_call = pl.pallas_call(
    _final_body,
    grid=(GRID,),
    in_specs=[
        pl.BlockSpec((R, D), lambda i: (i, 0)),
        pl.BlockSpec((1, R, D), lambda i: (0, i, 0)),
        pl.BlockSpec((1, R, D), lambda i: (1, i, 0)),
        pl.BlockSpec((R, 1), lambda i: (i, 0)),
        pl.BlockSpec((1, D), lambda i: (0, 0)),
        pl.BlockSpec((1, D), lambda i: (0, 0)),
        pl.BlockSpec((1, D), lambda i: (0, 0)),
        pl.BlockSpec((D, D_DENSE), lambda i: (0, 0)),
        pl.BlockSpec((1, D_DENSE), lambda i: (0, 0)),
        pl.BlockSpec((D_DENSE, D), lambda i: (0, 0)),
        pl.BlockSpec((1, D), lambda i: (0, 0)),
        pl.BlockSpec((D, D_OUT), lambda i: (0, 0)),
        pl.BlockSpec((1, D_OUT), lambda i: (0, 0)),
    ],
    out_specs=[
        pl.BlockSpec((R, D), lambda i: (i, 0)),
        pl.BlockSpec((R, D_OUT), lambda i: (i, 0)),
    ],
    out_shape=[
        jax.ShapeDtypeStruct((N, D), jnp.float32),
        jax.ShapeDtypeStruct((N, D_OUT), jnp.float32),
    ],
)


def kernel(x, edge_index, edge_attr, batch, W1, b1, g1, be1, W2, b2, g2, be2,
           W3, b3, g3, be3, Wp1, bp1, Wp2, bp2, Wp3, bp3):
    ei_pk = edge_index.reshape(2, NW, NCHUNK, C)

    deg_parts = _deg_parts(ei_pk, jnp.zeros((SLAB,), jnp.float32))
    dis = _dis_call(deg_parts)

    zslab = jnp.zeros((SLAB, D), jnp.float32)

    b1r, g1r, be1r = b1.reshape(1, D), g1.reshape(1, D), be1.reshape(1, D)
    b2r, g2r, be2r = b2.reshape(1, D), g2.reshape(1, D), be2.reshape(1, D)
    b3r, g3r, be3r = b3.reshape(1, D), g3.reshape(1, D), be3.reshape(1, D)

    hs = _stage0_call(x, W1, dis)
    parts = _edge_agg(hs, ei_pk, zslab).reshape(NC, NP, D)
    hs = _layer_call(hs, parts, parts, dis, b1r, g1r, be1r, W2)
    parts = _edge_agg(hs, ei_pk, zslab).reshape(NC, NP, D)
    hs = _layer_call(hs, parts, parts, dis, b2r, g2r, be2r, W3)
    parts = _edge_agg(hs, ei_pk, zslab).reshape(NC, NP, D)
    emb, logp = _final_call(
        hs, parts, parts, dis, b3r, g3r, be3r,
        Wp1, bp1.reshape(1, D_DENSE), Wp2, bp2.reshape(1, D),
        Wp3, bp3.reshape(1, D_OUT))
    return emb, logp


# speedup vs baseline: 1.0834x; 1.0328x over previous
import functools

import jax
import jax.numpy as jnp
from jax import lax
from jax.experimental import pallas as pl
from jax.experimental.pallas import tpu as pltpu
from jax.experimental.pallas import tpu_sc as plsc

N = 10000
NP = 10240
E = 320000
D = 128
D_DENSE = 256
D_OUT = 64
NC = 2
NS = 16
NW = NC * NS
EPW = E // NW
C = 125
NCHUNK = EPW // C
SLAB = NP // NS

_mesh = dict(core_axis_name="c", subcore_axis_name="s")


@functools.partial(
    pl.kernel,
    out_type=jax.ShapeDtypeStruct((NC, NP), jnp.float32),
    mesh=plsc.VectorSubcoreMesh(**_mesh),
    scratch_types=[
        pltpu.VMEM((NCHUNK, C), jnp.int32),
        pltpu.VMEM((128,), jnp.float32),
        pltpu.VMEM_SHARED((NP,), jnp.float32),
        pltpu.SemaphoreType.DMA,
        pltpu.SemaphoreType.DMA,
    ],
)
def _deg_parts(ei_hbm, zrow_hbm, out_hbm, dsti, onesb, acc, ss0, ss1):
    cid = lax.axis_index("c")
    sid = lax.axis_index("s")
    wid = sid * NC + cid
    ones16 = jnp.ones((16,), jnp.float32)
    for j in range(8):
        onesb[pl.ds(j * 16, 16)] = ones16
    ones = onesb.at[pl.ds(0, C)]
    pltpu.sync_copy(ei_hbm.at[1, wid], dsti)
    col0 = sid * SLAB
    pltpu.sync_copy(zrow_hbm, acc.at[pl.ds(col0, SLAB)])
    plsc.subcore_barrier()

    pltpu.async_copy(ones, acc.at[dsti.at[0]], ss0, add=True)
    pltpu.async_copy(ones, acc.at[dsti.at[1]], ss1, add=True)

    def cbody(t, _):
        i = 2 * t
        pltpu.make_async_copy(ones, acc.at[dsti.at[i]], ss0).wait()
        pltpu.async_copy(ones, acc.at[dsti.at[i + 2]], ss0, add=True)
        pltpu.make_async_copy(ones, acc.at[dsti.at[i + 1]], ss1).wait()
        pltpu.async_copy(ones, acc.at[dsti.at[i + 3]], ss1, add=True)
        return 0
    lax.fori_loop(0, NCHUNK // 2 - 1, cbody, 0)
    pltpu.make_async_copy(ones, acc.at[dsti.at[NCHUNK - 2]], ss0).wait()
    pltpu.make_async_copy(ones, acc.at[dsti.at[NCHUNK - 1]], ss1).wait()

    plsc.subcore_barrier()
    pltpu.sync_copy(acc.at[pl.ds(col0, SLAB)], out_hbm.at[cid, pl.ds(col0, SLAB)])


@functools.partial(
    pl.kernel,
    out_type=jax.ShapeDtypeStruct((NC * NP, D), jnp.float32),
    mesh=plsc.VectorSubcoreMesh(**_mesh),
    scratch_types=[
        pltpu.VMEM((NCHUNK, C), jnp.int32),
        pltpu.VMEM((C,), jnp.int32),
        pltpu.VMEM((C,), jnp.int32),
        pltpu.VMEM((C, D), jnp.float32),
        pltpu.VMEM((C, D), jnp.float32),
        pltpu.VMEM_SHARED((NP, D), jnp.float32),
        pltpu.SemaphoreType.DMA,
        pltpu.SemaphoreType.DMA,
        pltpu.SemaphoreType.DMA,
        pltpu.SemaphoreType.DMA,
        pltpu.SemaphoreType.DMA,
        pltpu.SemaphoreType.DMA,
    ],
)
def _edge_agg(hs_hbm, ei_hbm, zslab_hbm, out_hbm,
              dsti, srcb0, srcb1, rows0, rows1, acc, gs0, gs1, ss0, ss1, is0, is1):
    cid = lax.axis_index("c")
    sid = lax.axis_index("s")
    wid = sid * NC + cid
    r0 = sid * SLAB
    pltpu.sync_copy(ei_hbm.at[1, wid], dsti)
    pltpu.sync_copy(ei_hbm.at[0, wid, 0], srcb0)
    pltpu.async_copy(hs_hbm.at[srcb0], rows0, gs0)
    pltpu.sync_copy(ei_hbm.at[0, wid, 1], srcb1)
    pltpu.async_copy(hs_hbm.at[srcb1], rows1, gs1)
    pltpu.sync_copy(zslab_hbm, acc.at[pl.ds(r0, SLAB)])
    plsc.subcore_barrier()

    def cbody(t, _):
        i = 2 * t
        pltpu.make_async_copy(hs_hbm.at[srcb0], rows0, gs0).wait()
        pltpu.async_copy(ei_hbm.at[0, wid, i + 2], srcb0, is0)
        pltpu.async_copy(rows0, acc.at[dsti.at[i]], ss0, add=True)
        pltpu.make_async_copy(rows0, acc.at[dsti.at[i]], ss0).wait()
        pltpu.make_async_copy(ei_hbm.at[0, wid, i + 2], srcb0, is0).wait()
        pltpu.async_copy(hs_hbm.at[srcb0], rows0, gs0)
        pltpu.make_async_copy(hs_hbm.at[srcb1], rows1, gs1).wait()
        pltpu.async_copy(ei_hbm.at[0, wid, i + 3], srcb1, is1)
        pltpu.async_copy(rows1, acc.at[dsti.at[i + 1]], ss1, add=True)
        pltpu.make_async_copy(rows1, acc.at[dsti.at[i + 1]], ss1).wait()
        pltpu.make_async_copy(ei_hbm.at[0, wid, i + 3], srcb1, is1).wait()
        pltpu.async_copy(hs_hbm.at[srcb1], rows1, gs1)
        return 0
    lax.fori_loop(0, NCHUNK // 2 - 1, cbody, 0)

    i = NCHUNK - 2
    pltpu.make_async_copy(hs_hbm.at[srcb0], rows0, gs0).wait()
    pltpu.async_copy(rows0, acc.at[dsti.at[i]], ss0, add=True)
    pltpu.make_async_copy(rows0, acc.at[dsti.at[i]], ss0).wait()
    pltpu.make_async_copy(hs_hbm.at[srcb1], rows1, gs1).wait()
    pltpu.async_copy(rows1, acc.at[dsti.at[i + 1]], ss1, add=True)
    pltpu.make_async_copy(rows1, acc.at[dsti.at[i + 1]], ss1).wait()

    plsc.subcore_barrier()
    pltpu.sync_copy(acc.at[pl.ds(r0, SLAB)], out_hbm.at[pl.ds(cid * NP + r0, SLAB)])


R = 2000
GRID = N // R


def _dis_body(parts_ref, out_ref):
    p = parts_ref[...]
    out_ref[...] = jnp.transpose(lax.rsqrt(p[0:1, :] + p[1:2, :] + 1.0))


_dis_call = pl.pallas_call(
    _dis_body,
    out_shape=jax.ShapeDtypeStruct((NP, 1), jnp.float32),
)


def _stage0_body(x_ref, w_ref, dis_ref, hs_ref):
    hw = jnp.dot(x_ref[...], w_ref[...], preferred_element_type=jnp.float32)
    hs_ref[...] = hw * dis_ref[...]


_stage0_call = pl.pallas_call(
    _stage0_body,
    grid=(GRID,),
    in_specs=[
        pl.BlockSpec((R, D), lambda i: (i, 0)),
        pl.BlockSpec((D, D), lambda i: (0, 0)),
        pl.BlockSpec((R, 1), lambda i: (i, 0)),
    ],
    out_specs=pl.BlockSpec((R, D), lambda i: (i, 0)),
    out_shape=jax.ShapeDtypeStruct((N, D), jnp.float32),
)


def _ln_relu(conv, g, be):
    a = jnp.maximum(conv, 0.0)
    m = jnp.mean(a, axis=-1, keepdims=True)
    v = jnp.mean((a - m) ** 2, axis=-1, keepdims=True)
    return (a - m) * lax.rsqrt(v + 1e-5) * g + be


def _layer_body(hs_ref, p0_ref, p1_ref, dis_ref, b_ref, g_ref, be_ref, wn_ref,
                hsn_ref):
    dis = dis_ref[...]
    agg = p0_ref[0] + p1_ref[0]
    conv = dis * (agg + hs_ref[...]) + b_ref[...]
    h = _ln_relu(conv, g_ref[...], be_ref[...])
    hwn = jnp.dot(h, wn_ref[...], preferred_element_type=jnp.float32)
    hsn_ref[...] = hwn * dis


_layer_call = pl.pallas_call(
    _layer_body,
    grid=(GRID,),
    in_specs=[
        pl.BlockSpec((R, D), lambda i: (i, 0)),
        pl.BlockSpec((1, R, D), lambda i: (0, i, 0)),
        pl.BlockSpec((1, R, D), lambda i: (1, i, 0)),
        pl.BlockSpec((R, 1), lambda i: (i, 0)),
        pl.BlockSpec((1, D), lambda i: (0, 0)),
        pl.BlockSpec((1, D), lambda i: (0, 0)),
        pl.BlockSpec((1, D), lambda i: (0, 0)),
        pl.BlockSpec((D, D), lambda i: (0, 0)),
    ],
    out_specs=pl.BlockSpec((R, D), lambda i: (i, 0)),
    out_shape=jax.ShapeDtypeStruct((N, D), jnp.float32),
)


def _final_body(hs_ref, p0_ref, p1_ref, dis_ref, b_ref, g_ref, be_ref,
                wp1_ref, bp1_ref, wp2_ref, bp2_ref, wp3_ref, bp3_ref,
                emb_ref, logp_ref):
    dis = dis_ref[...]
    agg = p0_ref[0] + p1_ref[0]
    conv = dis * (agg + hs_ref[...]) + b_ref[...]
    emb_ref[...] = conv
    h = _ln_relu(conv, g_ref[...], be_ref[...])
    h = jnp.dot(h, wp1_ref[...], preferred_element_type=jnp.float32) + bp1_ref[...]
    h = jnp.dot(h, wp2_ref[...], preferred_element_type=jnp.float32) + bp2_ref[...]
    h = jnp.dot(h, wp3_ref[...], preferred_element_type=jnp.float32) + bp3_ref[...]
    mx = jnp.max(h, axis=-1, keepdims=True)
    sh = h - mx
    lse = jnp.log(jnp.sum(jnp.exp(sh), axis=-1, keepdims=True))
    logp_ref[...] = sh - lse


_final_call = pl.pallas_call(
    _final_body,
    grid=(GRID,),
    in_specs=[
        pl.BlockSpec((R, D), lambda i: (i, 0)),
        pl.BlockSpec((1, R, D), lambda i: (0, i, 0)),
        pl.BlockSpec((1, R, D), lambda i: (1, i, 0)),
        pl.BlockSpec((R, 1), lambda i: (i, 0)),
        pl.BlockSpec((1, D), lambda i: (0, 0)),
        pl.BlockSpec((1, D), lambda i: (0, 0)),
        pl.BlockSpec((1, D), lambda i: (0, 0)),
        pl.BlockSpec((D, D_DENSE), lambda i: (0, 0)),
        pl.BlockSpec((1, D_DENSE), lambda i: (0, 0)),
        pl.BlockSpec((D_DENSE, D), lambda i: (0, 0)),
        pl.BlockSpec((1, D), lambda i: (0, 0)),
        pl.BlockSpec((D, D_OUT), lambda i: (0, 0)),
        pl.BlockSpec((1, D_OUT), lambda i: (0, 0)),
    ],
    out_specs=[
        pl.BlockSpec((R, D), lambda i: (i, 0)),
        pl.BlockSpec((R, D_OUT), lambda i: (i, 0)),
    ],
    out_shape=[
        jax.ShapeDtypeStruct((N, D), jnp.float32),
        jax.ShapeDtypeStruct((N, D_OUT), jnp.float32),
    ],
)


def kernel(x, edge_index, edge_attr, batch, W1, b1, g1, be1, W2, b2, g2, be2,
           W3, b3, g3, be3, Wp1, bp1, Wp2, bp2, Wp3, bp3):
    ei_pk = edge_index.reshape(2, NW, NCHUNK, C)

    deg_parts = _deg_parts(ei_pk, jnp.zeros((SLAB,), jnp.float32))
    dis = _dis_call(deg_parts)

    zslab = jnp.zeros((SLAB, D), jnp.float32)

    b1r, g1r, be1r = b1.reshape(1, D), g1.reshape(1, D), be1.reshape(1, D)
    b2r, g2r, be2r = b2.reshape(1, D), g2.reshape(1, D), be2.reshape(1, D)
    b3r, g3r, be3r = b3.reshape(1, D), g3.reshape(1, D), be3.reshape(1, D)

    hs = _stage0_call(x, W1, dis)
    parts = _edge_agg(hs, ei_pk, zslab).reshape(NC, NP, D)
    hs = _layer_call(hs, parts, parts, dis, b1r, g1r, be1r, W2)
    parts = _edge_agg(hs, ei_pk, zslab).reshape(NC, NP, D)
    hs = _layer_call(hs, parts, parts, dis, b2r, g2r, be2r, W3)
    parts = _edge_agg(hs, ei_pk, zslab).reshape(NC, NP, D)
    emb, logp = _final_call(
        hs, parts, parts, dis, b3r, g3r, be3r,
        Wp1, bp1.reshape(1, D_DENSE), Wp2, bp2.reshape(1, D),
        Wp3, bp3.reshape(1, D_OUT))
    return emb, logp
